# baseline (speedup 1.0000x reference)
import jax
import jax.numpy as jnp
from jax.experimental import pallas as pl
from jax.experimental.pallas import tpu as pltpu
from jax.experimental.pallas import tpu_sc as plsc

VOCAB = 100000
DIM = 64
BATCH = 1024

N_BLK = 1024
NBUF = 7
NGRID = 14
NTILES = NGRID * NBUF
TAIL = VOCAB - (NTILES - 1) * N_BLK
GATHER_WINDOW = 128


def _gather_pairs_sc(emb2, pair_idx):
    idx2 = pair_idx.reshape(1, BATCH)
    mesh = plsc.VectorSubcoreMesh(core_axis_name="core",
                                  subcore_axis_name="subcore")

    @pl.kernel(out_type=jax.ShapeDtypeStruct((BATCH, 2 * DIM), emb2.dtype),
               mesh=mesh)
    def gather_kernel(emb_hbm, idx_hbm, out_hbm):
        def body(i_vmem, o_vmem):
            pltpu.sync_copy(emb_hbm.at[i_vmem.at[0]], o_vmem)

        pltpu.emit_pipeline(
            body,
            grid=(BATCH // GATHER_WINDOW,),
            in_specs=[pl.BlockSpec((1, GATHER_WINDOW),
                                   index_map=lambda i: (0, i))],
            out_specs=[pl.BlockSpec((GATHER_WINDOW, 2 * DIM),
                                    index_map=lambda i: (i, 0))],
            core_axis_name=("core", "subcore"),
            dimension_semantics=(pltpu.PARALLEL,),
        )(idx_hbm, out_hbm)

    return gather_kernel(emb2, idx2)


def _mm_body(x2_ref, par_ref, w_ref, b_ref, o_ref, x_s, tail_buf, *rest):
    bufs, sems = rest[:NBUF], rest[NBUF:]
    i = pl.program_id(0)

    @pl.when(i == 0)
    def _():
        par = par_ref[...]
        x = x2_ref[:, :DIM] * (1.0 - par) + x2_ref[:, DIM:] * par
        x_s[...] = x.astype(jnp.bfloat16)

    for k in range(NBUF):
        @pl.when(i > 0)
        def _(k=k):
            prev = ((i - 1) * NBUF + k) * N_BLK
            pltpu.make_async_copy(
                bufs[k], o_ref.at[:, pl.ds(prev, N_BLK)], sems[k]).wait()

        w_blk = w_ref[pl.ds(k * N_BLK, N_BLK), :].astype(jnp.bfloat16)
        acc = jax.lax.dot_general(
            x_s[...], w_blk,
            dimension_numbers=(((1,), (1,)), ((), ())),
            preferred_element_type=jnp.float32,
        )
        col = (i * NBUF + k) * N_BLK
        if k < NBUF - 1:
            bufs[k][...] = acc + b_ref[:, pl.ds(k * N_BLK, N_BLK)]
            pltpu.make_async_copy(
                bufs[k], o_ref.at[:, pl.ds(col, N_BLK)], sems[k]).start()
        else:
            @pl.when(i < NGRID - 1)
            def _(k=k, col=col, acc=acc):
                bufs[k][...] = acc + b_ref[:, pl.ds(k * N_BLK, N_BLK)]
                pltpu.make_async_copy(
                    bufs[k], o_ref.at[:, pl.ds(col, N_BLK)], sems[k]).start()

            @pl.when(i == NGRID - 1)
            def _(k=k, acc=acc):
                full = acc + b_ref[:, pl.ds(k * N_BLK, N_BLK)]
                tail_buf[...] = full[:, :TAIL]
                pltpu.make_async_copy(
                    tail_buf,
                    o_ref.at[:, pl.ds((NTILES - 1) * N_BLK, TAIL)],
                    sems[k]).start()

    @pl.when(i == NGRID - 1)
    def _():
        for k in range(NBUF - 1):
            col = ((NGRID - 1) * NBUF + k) * N_BLK
            pltpu.make_async_copy(
                bufs[k], o_ref.at[:, pl.ds(col, N_BLK)], sems[k]).wait()
        pltpu.make_async_copy(
            tail_buf,
            o_ref.at[:, pl.ds((NTILES - 1) * N_BLK, TAIL)],
            sems[NBUF - 1]).wait()


def _project_tc(x2, par, W, b2):
    return pl.pallas_call(
        _mm_body,
        grid=(NGRID,),
        in_specs=[
            pl.BlockSpec((BATCH, 2 * DIM), lambda i: (0, 0)),
            pl.BlockSpec((BATCH, 1), lambda i: (0, 0)),
            pl.BlockSpec((NBUF * N_BLK, DIM), lambda i: (i, 0)),
            pl.BlockSpec((1, NBUF * N_BLK), lambda i: (0, i)),
        ],
        out_specs=pl.BlockSpec(memory_space=pl.ANY),
        out_shape=jax.ShapeDtypeStruct((BATCH, VOCAB), jnp.float32),
        scratch_shapes=(
            [pltpu.VMEM((BATCH, DIM), jnp.bfloat16)]
            + [pltpu.VMEM((BATCH, TAIL), jnp.float32)]
            + [pltpu.VMEM((BATCH, N_BLK), jnp.float32) for _ in range(NBUF)]
            + [pltpu.SemaphoreType.DMA for _ in range(NBUF)]
        ),
    )(x2, par, W, b2)


def kernel(context_word, emb, W, b):
    idx = context_word.astype(jnp.int32)
    emb2 = emb.reshape(VOCAB // 2, 2 * DIM)
    x2 = _gather_pairs_sc(emb2, idx >> 1)
    par = (idx & 1).astype(jnp.float32).reshape(BATCH, 1)
    return _project_tc(x2, par, W, b.reshape(1, VOCAB))
